# Initial kernel scaffold; baseline (speedup 1.0000x reference)
import jax
import jax.numpy as jnp
from jax.experimental import pallas as pl


def kernel(h, edge_index, We1, be1, We2, be2, We3, be3, Wn1, bn1, Wn2, bn2):
    raise NotImplementedError("write your pallas kernel here")


# speedup vs baseline: 2.3493x; 2.3493x over previous
import functools

import jax
import jax.numpy as jnp
from jax import lax
from jax.experimental import pallas as pl
from jax.experimental.pallas import tpu as pltpu
from jax.experimental.pallas import tpu_sc as plsc

V = 10000
E = 320000
D = 128

NC = 2
NS = 16
NW = NC * NS
EPW = E // NW
CH = 80
NCH = EPW // CH
VP = 10240
VPS = VP // NS
ZR = 128


def _proj_body(h_ref, wa_ref, wb_ref, be1_ref, p_ref, q_ref):
    h = h_ref[...]
    p_ref[...] = jnp.dot(h, wa_ref[...], preferred_element_type=jnp.float32)
    q_ref[...] = (jnp.dot(h, wb_ref[...], preferred_element_type=jnp.float32)
                  + be1_ref[...])


def _edge_mlp_body(ps_ref, qd_ref, w2_ref, b2_ref, w3_ref, b3_ref, msg_ref):
    x1 = jnp.maximum(ps_ref[...] + qd_ref[...], 0.0)
    x2 = jnp.maximum(
        jnp.dot(x1, w2_ref[...], preferred_element_type=jnp.float32)
        + b2_ref[...], 0.0)
    msg_ref[...] = (jnp.dot(x2, w3_ref[...], preferred_element_type=jnp.float32)
                    + b3_ref[...])


def _node_mlp_body(h_ref, a0_ref, a1_ref, wna_ref, wnb_ref, bn1_ref,
                   wn2_ref, bn2_ref, out_ref):
    agg = a0_ref[0] + a1_ref[0]
    y = jnp.maximum(
        jnp.dot(h_ref[...], wna_ref[...], preferred_element_type=jnp.float32)
        + jnp.dot(agg, wnb_ref[...], preferred_element_type=jnp.float32)
        + bn1_ref[...], 0.0)
    out_ref[...] = (jnp.dot(y, wn2_ref[...], preferred_element_type=jnp.float32)
                    + bn2_ref[...])


def _full(shape):
    return pl.BlockSpec(shape, lambda i: (0,) * len(shape))


def _gather_body(p_hbm, q_hbm, src_hbm, dst_hbm, ps_hbm, qd_hbm,
                 sidx, didx, pbuf, qbuf, sem):
    c = lax.axis_index("c")
    s = lax.axis_index("s")
    wid = s * NC + c
    base = wid * EPW

    def step(i, carry):
        off = base + i * CH
        pltpu.sync_copy(src_hbm.at[pl.ds(off, CH)], sidx)
        pltpu.sync_copy(dst_hbm.at[pl.ds(off, CH)], didx)
        cp1 = pltpu.async_copy(p_hbm.at[sidx], pbuf, sem)
        cp2 = pltpu.async_copy(q_hbm.at[didx], qbuf, sem)
        cp1.wait()
        cp2.wait()
        pltpu.sync_copy(pbuf, ps_hbm.at[pl.ds(off, CH)])
        pltpu.sync_copy(qbuf, qd_hbm.at[pl.ds(off, CH)])
        return carry

    lax.fori_loop(0, NCH, step, 0)


def _scatter_body(msg_hbm, dst_hbm, out_hbm, didx, mbuf, zbuf, agg_sh):
    c = lax.axis_index("c")
    s = lax.axis_index("s")
    wid = s * NC + c
    base = wid * EPW

    zero = jnp.zeros((16,), jnp.float32)

    def zrow(i, carry):
        for j in range(D // 16):
            zbuf[i, pl.ds(j * 16, 16)] = zero
        return carry

    lax.fori_loop(0, ZR, zrow, 0)
    for k in range(VPS // ZR):
        pltpu.sync_copy(zbuf, agg_sh.at[pl.ds(s * VPS + k * ZR, ZR)])
    plsc.subcore_barrier()

    def step(i, carry):
        off = base + i * CH
        pltpu.sync_copy(dst_hbm.at[pl.ds(off, CH)], didx)
        pltpu.sync_copy(msg_hbm.at[pl.ds(off, CH)], mbuf)
        pltpu.sync_copy(mbuf, agg_sh.at[didx], add=True)
        return carry

    lax.fori_loop(0, NCH, step, 0)
    plsc.subcore_barrier()
    pltpu.sync_copy(agg_sh.at[pl.ds(s * VPS, VPS)],
                    out_hbm.at[c, pl.ds(s * VPS, VPS)])


@jax.jit
def kernel(h, edge_index, We1, be1, We2, be2, We3, be3, Wn1, bn1, Wn2, bn2):
    src = edge_index[0].astype(jnp.int32)
    dst = edge_index[1].astype(jnp.int32)
    be1r = be1.reshape(1, D)
    be2r = be2.reshape(1, D)
    be3r = be3.reshape(1, D)
    bn1r = bn1.reshape(1, D)
    bn2r = bn2.reshape(1, D)

    BV = 1000
    p, q = pl.pallas_call(
        _proj_body,
        grid=(V // BV,),
        in_specs=[
            pl.BlockSpec((BV, D), lambda i: (i, 0)),
            _full((D, D)), _full((D, D)), _full((1, D)),
        ],
        out_specs=[
            pl.BlockSpec((BV, D), lambda i: (i, 0)),
            pl.BlockSpec((BV, D), lambda i: (i, 0)),
        ],
        out_shape=[
            jax.ShapeDtypeStruct((V, D), jnp.float32),
            jax.ShapeDtypeStruct((V, D), jnp.float32),
        ],
    )(h, We1[:D], We1[D:], be1r)

    mesh = plsc.VectorSubcoreMesh(core_axis_name="c", subcore_axis_name="s")
    ps, qd = pl.kernel(
        _gather_body,
        out_type=[
            jax.ShapeDtypeStruct((E, D), jnp.float32),
            jax.ShapeDtypeStruct((E, D), jnp.float32),
        ],
        mesh=mesh,
        scratch_types=[
            pltpu.VMEM((CH,), jnp.int32),
            pltpu.VMEM((CH,), jnp.int32),
            pltpu.VMEM((CH, D), jnp.float32),
            pltpu.VMEM((CH, D), jnp.float32),
            pltpu.SemaphoreType.DMA,
        ],
    )(p, q, src, dst)

    BE = 512
    msg = pl.pallas_call(
        _edge_mlp_body,
        grid=(E // BE,),
        in_specs=[
            pl.BlockSpec((BE, D), lambda i: (i, 0)),
            pl.BlockSpec((BE, D), lambda i: (i, 0)),
            _full((D, D)), _full((1, D)), _full((D, D)), _full((1, D)),
        ],
        out_specs=pl.BlockSpec((BE, D), lambda i: (i, 0)),
        out_shape=jax.ShapeDtypeStruct((E, D), jnp.float32),
    )(ps, qd, We2, be2r, We3, be3r)

    aggp = pl.kernel(
        _scatter_body,
        out_type=jax.ShapeDtypeStruct((NC, VP, D), jnp.float32),
        mesh=mesh,
        scratch_types=[
            pltpu.VMEM((CH,), jnp.int32),
            pltpu.VMEM((CH, D), jnp.float32),
            pltpu.VMEM((ZR, D), jnp.float32),
            pltpu.VMEM_SHARED((VP, D), jnp.float32),
        ],
    )(msg, dst)

    out = pl.pallas_call(
        _node_mlp_body,
        grid=(V // BV,),
        in_specs=[
            pl.BlockSpec((BV, D), lambda i: (i, 0)),
            pl.BlockSpec((1, BV, D), lambda i: (0, i, 0)),
            pl.BlockSpec((1, BV, D), lambda i: (1, i, 0)),
            _full((D, D)), _full((D, D)), _full((1, D)),
            _full((D, D)), _full((1, D)),
        ],
        out_specs=pl.BlockSpec((BV, D), lambda i: (i, 0)),
        out_shape=jax.ShapeDtypeStruct((V, D), jnp.float32),
    )(h, aggp, aggp, Wn1[:D], Wn1[D:], bn1r, Wn2, bn2r)
    return out
